# baseline (speedup 1.0000x reference)
import jax
import jax.numpy as jnp
from jax import lax
from jax.experimental import pallas as pl
from jax.experimental.pallas import tpu as pltpu
from jax.experimental.pallas import tpu_sc as plsc

_N = 10000
_NP = 10240
_ECH = 128
_NCH = 84
_EPT = _NCH * _ECH
_EPAD = 32 * _EPT
_PADN = 10008
_MESH = plsc.VectorSubcoreMesh(core_axis_name="c", subcore_axis_name="s")


def _prelude(z_hbm, acc, s, width):
    nps = _NP // 16
    slab = s * nps
    pltpu.sync_copy(z_hbm.at[pl.ds(slab, nps)], acc.at[pl.ds(slab, nps)])
    plsc.subcore_barrier()
    return slab, nps


def _epilogue(acc, out_hbm, c, slab, nps):
    plsc.subcore_barrier()
    pltpu.sync_copy(acc.at[pl.ds(slab, nps)], out_hbm.at[c, pl.ds(slab, nps)])


def _apass_body(p_hbm, src_hbm, dst_hbm, zero_hbm, out_hbm,
                sidx0, sidx1, didx0, didx1, didx2, didx3, rows0, rows1,
                ss0, ss1, sd0, sd1, sd2, sd3, gr0, gr1, ws0, ws1, acc):
    c = lax.axis_index("c")
    s = lax.axis_index("s")
    slab, nps = _prelude(zero_hbm, acc, s, 128)
    base0 = (c * 16 + s) * _EPT
    sidx = (sidx0, sidx1)
    didx = (didx0, didx1, didx2, didx3)
    rows = (rows0, rows1)
    ssem = (ss0, ss1)
    dsem = (sd0, sd1, sd2, sd3)
    rsem = (gr0, gr1)
    wsem = (ws0, ws1)

    def idx_load(k, b2, b4):
        pltpu.async_copy(src_hbm.at[pl.ds(base0 + k * _ECH, _ECH)], sidx[b2], ssem[b2])
        pltpu.async_copy(dst_hbm.at[pl.ds(base0 + k * _ECH, _ECH)], didx[b4], dsem[b4])

    def idx_wait(b2, b4):
        pltpu.make_async_copy(src_hbm.at[pl.ds(0, _ECH)], sidx[b2], ssem[b2]).wait()
        pltpu.make_async_copy(dst_hbm.at[pl.ds(0, _ECH)], didx[b4], dsem[b4]).wait()

    def gath(b2):
        pltpu.async_copy(p_hbm.at[sidx[b2]], rows[b2], rsem[b2])

    def gath_wait(b2):
        pltpu.make_async_copy(p_hbm.at[sidx[b2]], rows[b2], rsem[b2]).wait()

    def scat(b2, b4):
        pltpu.async_copy(rows[b2], acc.at[didx[b4]], wsem[b2], add=True)

    def scat_wait(b2):
        pltpu.make_async_copy(rows[b2], acc.at[didx[0]], wsem[b2]).wait()

    idx_load(0, 0, 0)
    idx_wait(0, 0)
    gath(0)
    idx_load(1, 1, 1)

    def quad(kk, carry):
        for b in (0, 1, 2, 3):
            k = 4 * kk + b
            b2 = b % 2
            nb2 = 1 - b2
            gath_wait(b2)

            @pl.when((k + 1 < _NCH) & (k >= 1))
            def _():
                scat_wait(nb2)

            @pl.when(k + 1 < _NCH)
            def _():
                idx_wait(nb2, (b + 1) % 4)
                gath(nb2)

            scat(b2, b)

            @pl.when(k + 2 < _NCH)
            def _():
                idx_load(k + 2, b2, (b + 2) % 4)
        return carry

    lax.fori_loop(0, _NCH // 4, quad, 0)
    scat_wait(0)
    scat_wait(1)
    _epilogue(acc, out_hbm, c, slab, nps)


_apass = pl.kernel(
    _apass_body,
    out_type=jax.ShapeDtypeStruct((2, _NP, 128), jnp.float32),
    mesh=_MESH,
    scratch_types=[
        pltpu.VMEM((_ECH,), jnp.int32), pltpu.VMEM((_ECH,), jnp.int32),
        pltpu.VMEM((_ECH,), jnp.int32), pltpu.VMEM((_ECH,), jnp.int32),
        pltpu.VMEM((_ECH,), jnp.int32), pltpu.VMEM((_ECH,), jnp.int32),
        pltpu.VMEM((_ECH, 128), jnp.float32), pltpu.VMEM((_ECH, 128), jnp.float32),
        pltpu.SemaphoreType.DMA, pltpu.SemaphoreType.DMA,
        pltpu.SemaphoreType.DMA, pltpu.SemaphoreType.DMA,
        pltpu.SemaphoreType.DMA, pltpu.SemaphoreType.DMA,
        pltpu.SemaphoreType.DMA, pltpu.SemaphoreType.DMA,
        pltpu.SemaphoreType.DMA, pltpu.SemaphoreType.DMA,
        pltpu.VMEM_SHARED((_NP, 128), jnp.float32),
    ],
)


def _gatden_body(as_hbm, ad_hbm, src_hbm, dst_hbm, m_hbm, zero_hbm,
                 out_hbm, ae_hbm,
                 sidx0, sidx1, didx0, didx1, asv0, asv1, adv0, adv1,
                 scv, mv,
                 ss0, ss1, sd0, sd1, ga0, ga1, gb0, gb1, acc):
    c = lax.axis_index("c")
    s = lax.axis_index("s")
    slab, nps = _prelude(zero_hbm, acc, s, 128)
    pltpu.sync_copy(m_hbm, mv)
    base0 = (c * 16 + s) * _EPT
    sidx = (sidx0, sidx1)
    didx = (didx0, didx1)
    asv = (asv0, asv1)
    adv = (adv0, adv1)
    ssem = (ss0, ss1)
    dsem = (sd0, sd1)
    asem = (ga0, ga1)
    bsem = (gb0, gb1)
    iota = lax.iota(jnp.int32, 16)
    mvec = mv[...]

    def zrow(e, carry):
        for cc in range(8):
            scv[e, pl.ds(cc * 16, 16)] = jnp.zeros((16,), jnp.float32)
        return carry

    lax.fori_loop(0, _ECH, zrow, 0)

    def idx_load(k, b):
        pltpu.async_copy(src_hbm.at[pl.ds(base0 + k * _ECH, _ECH)], sidx[b], ssem[b])
        pltpu.async_copy(dst_hbm.at[pl.ds(base0 + k * _ECH, _ECH)], didx[b], dsem[b])

    def idx_wait(b):
        pltpu.make_async_copy(src_hbm.at[pl.ds(0, _ECH)], sidx[b], ssem[b]).wait()
        pltpu.make_async_copy(dst_hbm.at[pl.ds(0, _ECH)], didx[b], dsem[b]).wait()

    def gath(b):
        pltpu.async_copy(as_hbm.at[sidx[b]], asv[b], asem[b])
        pltpu.async_copy(ad_hbm.at[didx[b]], adv[b], bsem[b])

    def gath_wait(b):
        pltpu.make_async_copy(as_hbm.at[sidx[b]], asv[b], asem[b]).wait()
        pltpu.make_async_copy(ad_hbm.at[didx[b]], adv[b], bsem[b]).wait()

    idx_load(0, 0)
    idx_wait(0)
    gath(0)
    idx_load(1, 1)

    def pair(kk, carry):
        for b in (0, 1):
            k = 2 * kk + b
            nb = 1 - b
            gath_wait(b)

            @pl.when(k + 1 < _NCH)
            def _():
                idx_wait(nb)
                gath(nb)

            for j in range(_ECH // 16):
                a = asv[b][pl.ds(j * 16, 16)] + adv[b][pl.ds(j * 16, 16)]
                a = jnp.where(a > 0.0, a, 0.2 * a)
                av = jnp.exp(jnp.minimum(a - mvec, 50.0))
                asv[b][pl.ds(j * 16, 16)] = av
                for ee in range(16):
                    scv[j * 16 + ee, pl.ds(0, 16)] = jnp.where(
                        iota == 0, av[ee], jnp.where(iota == 1, 1.0, 0.0))
            pltpu.sync_copy(asv[b], ae_hbm.at[pl.ds(base0 + k * _ECH, _ECH)])
            pltpu.sync_copy(scv, acc.at[didx[b]], add=True)

            @pl.when(k + 2 < _NCH)
            def _():
                idx_load(k + 2, b)
        return carry

    lax.fori_loop(0, _NCH // 2, pair, 0)
    _epilogue(acc, out_hbm, c, slab, nps)


_gatden = pl.kernel(
    _gatden_body,
    out_type=(jax.ShapeDtypeStruct((2, _NP, 128), jnp.float32),
              jax.ShapeDtypeStruct((_EPAD,), jnp.float32)),
    mesh=_MESH,
    scratch_types=[
        pltpu.VMEM((_ECH,), jnp.int32), pltpu.VMEM((_ECH,), jnp.int32),
        pltpu.VMEM((_ECH,), jnp.int32), pltpu.VMEM((_ECH,), jnp.int32),
        pltpu.VMEM((_ECH,), jnp.float32), pltpu.VMEM((_ECH,), jnp.float32),
        pltpu.VMEM((_ECH,), jnp.float32), pltpu.VMEM((_ECH,), jnp.float32),
        pltpu.VMEM((_ECH, 128), jnp.float32),
        pltpu.VMEM((16,), jnp.float32),
        pltpu.SemaphoreType.DMA, pltpu.SemaphoreType.DMA,
        pltpu.SemaphoreType.DMA, pltpu.SemaphoreType.DMA,
        pltpu.SemaphoreType.DMA, pltpu.SemaphoreType.DMA,
        pltpu.SemaphoreType.DMA, pltpu.SemaphoreType.DMA,
        pltpu.VMEM_SHARED((_NP, 128), jnp.float32),
    ],
)


def _gatnum_body(xw_hbm, den_hbm, ae_hbm, src_hbm, dst_hbm, zero_hbm, out_hbm,
                 sidx0, sidx1, didx0, didx1, aev0, aev1, dnv0, dnv1,
                 rows0, rows1,
                 ss0, ss1, sd0, sd1, ga0, ga1, gb0, gb1, gr0, gr1, acc):
    c = lax.axis_index("c")
    s = lax.axis_index("s")
    slab, nps = _prelude(zero_hbm, acc, s, 128)
    base0 = (c * 16 + s) * _EPT
    sidx = (sidx0, sidx1)
    didx = (didx0, didx1)
    aev = (aev0, aev1)
    dnv = (dnv0, dnv1)
    rows = (rows0, rows1)
    ssem = (ss0, ss1)
    dsem = (sd0, sd1)
    asem = (ga0, ga1)
    bsem = (gb0, gb1)
    rsem = (gr0, gr1)

    def idx_load(k, b):
        pltpu.async_copy(src_hbm.at[pl.ds(base0 + k * _ECH, _ECH)], sidx[b], ssem[b])
        pltpu.async_copy(dst_hbm.at[pl.ds(base0 + k * _ECH, _ECH)], didx[b], dsem[b])

    def idx_wait(b):
        pltpu.make_async_copy(src_hbm.at[pl.ds(0, _ECH)], sidx[b], ssem[b]).wait()
        pltpu.make_async_copy(dst_hbm.at[pl.ds(0, _ECH)], didx[b], dsem[b]).wait()

    def gath(k, b):
        pltpu.async_copy(ae_hbm.at[pl.ds(base0 + k * _ECH, _ECH)], aev[b], asem[b])
        pltpu.async_copy(den_hbm.at[didx[b]], dnv[b], bsem[b])
        pltpu.async_copy(xw_hbm.at[sidx[b]], rows[b], rsem[b])

    def gath_wait(b):
        pltpu.make_async_copy(ae_hbm.at[pl.ds(0, _ECH)], aev[b], asem[b]).wait()
        pltpu.make_async_copy(den_hbm.at[didx[b]], dnv[b], bsem[b]).wait()
        pltpu.make_async_copy(xw_hbm.at[sidx[b]], rows[b], rsem[b]).wait()

    idx_load(0, 0)
    idx_wait(0)
    gath(0, 0)
    idx_load(1, 1)

    def pair(kk, carry):
        for b in (0, 1):
            k = 2 * kk + b
            nb = 1 - b
            gath_wait(b)

            @pl.when(k + 1 < _NCH)
            def _():
                idx_wait(nb)
                gath(k + 1, nb)

            def grp(j, carry2):
                av = aev[b][pl.ds(j * 16, 16)] / dnv[b][pl.ds(j * 16, 16)]
                for ee in range(16):
                    e = j * 16 + ee
                    w = av[ee]
                    for cc in range(8):
                        rows[b][e, pl.ds(cc * 16, 16)] = (
                            rows[b][e, pl.ds(cc * 16, 16)] * w)
                return carry2

            lax.fori_loop(0, _ECH // 16, grp, 0)
            pltpu.sync_copy(rows[b], acc.at[didx[b]], add=True)

            @pl.when(k + 2 < _NCH)
            def _():
                idx_load(k + 2, b)
        return carry

    lax.fori_loop(0, _NCH // 2, pair, 0)
    _epilogue(acc, out_hbm, c, slab, nps)


_gatnum = pl.kernel(
    _gatnum_body,
    out_type=jax.ShapeDtypeStruct((2, _NP, 128), jnp.float32),
    mesh=_MESH,
    scratch_types=[
        pltpu.VMEM((_ECH,), jnp.int32), pltpu.VMEM((_ECH,), jnp.int32),
        pltpu.VMEM((_ECH,), jnp.int32), pltpu.VMEM((_ECH,), jnp.int32),
        pltpu.VMEM((_ECH,), jnp.float32), pltpu.VMEM((_ECH,), jnp.float32),
        pltpu.VMEM((_ECH,), jnp.float32), pltpu.VMEM((_ECH,), jnp.float32),
        pltpu.VMEM((_ECH, 128), jnp.float32), pltpu.VMEM((_ECH, 128), jnp.float32),
        pltpu.SemaphoreType.DMA, pltpu.SemaphoreType.DMA,
        pltpu.SemaphoreType.DMA, pltpu.SemaphoreType.DMA,
        pltpu.SemaphoreType.DMA, pltpu.SemaphoreType.DMA,
        pltpu.SemaphoreType.DMA, pltpu.SemaphoreType.DMA,
        pltpu.SemaphoreType.DMA, pltpu.SemaphoreType.DMA,
        pltpu.VMEM_SHARED((_NP, 128), jnp.float32),
    ],
)


def _erelu_body(u_hbm, v_hbm, src_hbm, dst_hbm, r_hbm,
                sidx0, sidx1, didx0, didx1, ru0, ru1, rv0, rv1,
                ss0, ss1, sd0, sd1, gu0, gu1, gv0, gv1):
    c = lax.axis_index("c")
    s = lax.axis_index("s")
    base0 = (c * 16 + s) * _EPT
    sidx = (sidx0, sidx1)
    didx = (didx0, didx1)
    ru = (ru0, ru1)
    rv = (rv0, rv1)
    ssem = (ss0, ss1)
    dsem = (sd0, sd1)
    usem = (gu0, gu1)
    vsem = (gv0, gv1)

    def idx_load(k, b):
        pltpu.async_copy(src_hbm.at[pl.ds(base0 + k * _ECH, _ECH)], sidx[b], ssem[b])
        pltpu.async_copy(dst_hbm.at[pl.ds(base0 + k * _ECH, _ECH)], didx[b], dsem[b])

    def idx_wait(b):
        pltpu.make_async_copy(src_hbm.at[pl.ds(0, _ECH)], sidx[b], ssem[b]).wait()
        pltpu.make_async_copy(dst_hbm.at[pl.ds(0, _ECH)], didx[b], dsem[b]).wait()

    def gath(b):
        pltpu.async_copy(u_hbm.at[sidx[b]], ru[b], usem[b])
        pltpu.async_copy(v_hbm.at[didx[b]], rv[b], vsem[b])

    def gath_wait(b):
        pltpu.make_async_copy(u_hbm.at[sidx[b]], ru[b], usem[b]).wait()
        pltpu.make_async_copy(v_hbm.at[didx[b]], rv[b], vsem[b]).wait()

    idx_load(0, 0)
    idx_wait(0)
    gath(0)
    idx_load(1, 1)

    def pair(kk, carry):
        for b in (0, 1):
            k = 2 * kk + b
            nb = 1 - b
            gath_wait(b)

            @pl.when(k + 1 < _NCH)
            def _():
                idx_wait(nb)
                gath(nb)

            def erow(e, carry2):
                for cc in range(8):
                    d = ru[b][e, pl.ds(cc * 16, 16)] - rv[b][e, pl.ds(cc * 16, 16)]
                    ru[b][e, pl.ds(cc * 16, 16)] = jnp.maximum(d, 0.0)
                return carry2

            lax.fori_loop(0, _ECH, erow, 0)
            pltpu.sync_copy(ru[b], r_hbm.at[pl.ds(base0 + k * _ECH, _ECH)])

            @pl.when(k + 2 < _NCH)
            def _():
                idx_load(k + 2, b)
        return carry

    lax.fori_loop(0, _NCH // 2, pair, 0)


_erelu = pl.kernel(
    _erelu_body,
    out_type=jax.ShapeDtypeStruct((_EPAD, 128), jnp.float32),
    mesh=_MESH,
    scratch_types=[
        pltpu.VMEM((_ECH,), jnp.int32), pltpu.VMEM((_ECH,), jnp.int32),
        pltpu.VMEM((_ECH,), jnp.int32), pltpu.VMEM((_ECH,), jnp.int32),
        pltpu.VMEM((_ECH, 128), jnp.float32), pltpu.VMEM((_ECH, 128), jnp.float32),
        pltpu.VMEM((_ECH, 128), jnp.float32), pltpu.VMEM((_ECH, 128), jnp.float32),
        pltpu.SemaphoreType.DMA, pltpu.SemaphoreType.DMA,
        pltpu.SemaphoreType.DMA, pltpu.SemaphoreType.DMA,
        pltpu.SemaphoreType.DMA, pltpu.SemaphoreType.DMA,
        pltpu.SemaphoreType.DMA, pltpu.SemaphoreType.DMA,
    ],
)


_RB = 1024
_EB = 2048


def _rows_spec(width):
    return pl.BlockSpec((_RB, width), lambda i: (i, 0))


def _full_spec(r, c):
    return pl.BlockSpec((r, c), lambda i: (0, 0))


def _halves_spec(width):
    return pl.BlockSpec((2, _RB, width), lambda i: (0, i, 0))


def _tc_pre_body(x_ref, p_ref, wx_ref, wp_ref, b_ref, u_ref, v_ref):
    vv = jnp.dot(p_ref[...], wp_ref[...], preferred_element_type=jnp.float32)
    u_ref[...] = (jnp.dot(x_ref[...], wx_ref[...],
                          preferred_element_type=jnp.float32)
                  + vv + b_ref[...])
    v_ref[...] = vv


def _tc_pre(xp, posp, wx, wp, b1):
    return pl.pallas_call(
        _tc_pre_body,
        grid=(_NP // _RB,),
        in_specs=[_rows_spec(128), _rows_spec(8), _full_spec(128, 128),
                  _full_spec(8, 128), _full_spec(1, 128)],
        out_specs=[_rows_spec(128), _rows_spec(128)],
        out_shape=[jax.ShapeDtypeStruct((_NP, 128), jnp.float32),
                   jax.ShapeDtypeStruct((_NP, 128), jnp.float32)],
    )(xp, posp, wx, wp, b1)


def _tc_edgemm_body(r_ref, w_ref, o_ref):
    o_ref[...] = jnp.dot(r_ref[...], w_ref[...],
                         preferred_element_type=jnp.float32).astype(jnp.bfloat16)


def _tc_edgemm(r, w):
    return pl.pallas_call(
        _tc_edgemm_body,
        grid=(_EPAD // _EB,),
        in_specs=[pl.BlockSpec((_EB, 128), lambda i: (i, 0)),
                  _full_spec(128, 128)],
        out_specs=pl.BlockSpec((_EB, 128), lambda i: (i, 0)),
        out_shape=jax.ShapeDtypeStruct((_EPAD, 128), jnp.bfloat16),
    )(r, w)


def _tc_mlp_body(agg_ref, w1_ref, b1_ref, w2_ref, b2_ref, w3_ref, b3_ref,
                 wg_ref, asr_ref, adr_ref, xw_ref, as_ref, ad_ref, mx_ref):
    g = jax.nn.relu(jnp.dot(agg_ref[...], w1_ref[...],
                            preferred_element_type=jnp.float32) + b1_ref[...])
    g = jax.nn.relu(jnp.dot(g, w2_ref[...],
                            preferred_element_type=jnp.float32) + b2_ref[...])
    h0 = jnp.dot(g, w3_ref[...], preferred_element_type=jnp.float32) + b3_ref[...]
    xw = jnp.dot(h0, wg_ref[...], preferred_element_type=jnp.float32)
    xw_ref[...] = xw
    a_s = jnp.sum(xw * asr_ref[...], axis=1, keepdims=True)
    a_d = jnp.sum(xw * adr_ref[...], axis=1, keepdims=True)
    as_ref[...] = a_s
    ad_ref[...] = a_d
    col = lax.broadcasted_iota(jnp.int32, (8, 128), 1)
    mx_ref[...] = jnp.where(col == 0, jnp.max(a_s),
                            jnp.where(col == 1, jnp.max(a_d), 0.0))


def _tc_mlp(aggp, w1, b1, w2, b2, w3, b3, wg, asr, adr):
    return pl.pallas_call(
        _tc_mlp_body,
        grid=(_NP // _RB,),
        in_specs=[_rows_spec(128), _full_spec(128, 64), _full_spec(1, 64),
                  _full_spec(64, 256), _full_spec(1, 256),
                  _full_spec(256, 128), _full_spec(1, 128),
                  _full_spec(128, 128), _full_spec(1, 128), _full_spec(1, 128)],
        out_specs=[_rows_spec(128),
                   pl.BlockSpec((_RB, 1), lambda i: (i, 0)),
                   pl.BlockSpec((_RB, 1), lambda i: (i, 0)),
                   pl.BlockSpec((8, 128), lambda i: (i, 0))],
        out_shape=[jax.ShapeDtypeStruct((_NP, 128), jnp.float32),
                   jax.ShapeDtypeStruct((_NP, 1), jnp.float32),
                   jax.ShapeDtypeStruct((_NP, 1), jnp.float32),
                   jax.ShapeDtypeStruct((_NP // _RB * 8, 128), jnp.float32)],
    )(aggp, w1, b1, w2, b2, w3, b3, wg, asr, adr)


def _tc_den_body(dh_ref, den_ref, deg_ref):
    s = dh_ref[0] + dh_ref[1]
    den_ref[...] = s[:, 0:1]
    deg_ref[...] = s[:, 1:2]


def _tc_den(dh):
    return pl.pallas_call(
        _tc_den_body,
        grid=(_NP // _RB,),
        in_specs=[_halves_spec(128)],
        out_specs=[pl.BlockSpec((_RB, 1), lambda i: (i, 0)),
                   pl.BlockSpec((_RB, 1), lambda i: (i, 0))],
        out_shape=[jax.ShapeDtypeStruct((_NP, 1), jnp.float32),
                   jax.ShapeDtypeStruct((_NP, 1), jnp.float32)],
    )(dh)


def _tc_gatfin_body(nh_ref, deg_ref, b_ref, p_ref, dinv_ref):
    h1 = jax.nn.relu(nh_ref[0] + nh_ref[1] + b_ref[...])
    deg = deg_ref[...]
    dinv = jnp.where(deg > 0.0, lax.rsqrt(jnp.maximum(deg, 1e-30)), 0.0)
    dinv_ref[...] = dinv
    p_ref[...] = dinv * h1


def _tc_gatfin(nh, deg, gb):
    return pl.pallas_call(
        _tc_gatfin_body,
        grid=(_NP // _RB,),
        in_specs=[_halves_spec(128), pl.BlockSpec((_RB, 1), lambda i: (i, 0)),
                  _full_spec(1, 128)],
        out_specs=[_rows_spec(128), pl.BlockSpec((_RB, 1), lambda i: (i, 0))],
        out_shape=[jax.ShapeDtypeStruct((_NP, 128), jnp.float32),
                   jax.ShapeDtypeStruct((_NP, 1), jnp.float32)],
    )(nh, deg, gb)


def _tc_gcn_body(qh_ref, dinv_ref, w_ref, b_ref, p_ref):
    q = (qh_ref[0] + qh_ref[1]) * dinv_ref[...]
    hnew = jax.nn.relu(jnp.dot(q, w_ref[...],
                               preferred_element_type=jnp.float32) + b_ref[...])
    p_ref[...] = dinv_ref[...] * hnew


def _tc_gcn(qh, dinv, w, b):
    return pl.pallas_call(
        _tc_gcn_body,
        grid=(_NP // _RB,),
        in_specs=[_halves_spec(128), pl.BlockSpec((_RB, 1), lambda i: (i, 0)),
                  _full_spec(128, 128), _full_spec(1, 128)],
        out_specs=_rows_spec(128),
        out_shape=jax.ShapeDtypeStruct((_NP, 128), jnp.float32),
    )(qh, dinv, w, b)


def _tc_final_body(qh_ref, dinv_ref, w_ref, b_ref, o_ref):
    q = (qh_ref[0] + qh_ref[1]) * dinv_ref[...]
    o_ref[...] = jnp.dot(q, w_ref[...],
                         preferred_element_type=jnp.float32) + b_ref[...]


def _tc_final(qh, dinv, w, b):
    return pl.pallas_call(
        _tc_final_body,
        grid=(_NP // _RB,),
        in_specs=[_halves_spec(128), pl.BlockSpec((_RB, 1), lambda i: (i, 0)),
                  _full_spec(128, 128), _full_spec(1, 128)],
        out_specs=_rows_spec(128),
        out_shape=jax.ShapeDtypeStruct((_NP, 128), jnp.float32),
    )(qh, dinv, w, b)


def kernel(x, pos, edge_index, ln1_w, ln1_b, ln2_w, ln2_b, gn1_w, gn1_b, gn2_w, gn2_b, gn3_w, gn3_b, gat_w, gat_asrc, gat_adst, gat_b, gcn1_w, gcn1_b, gcn2_w, gcn2_b, out_w, out_b):
    n, d = x.shape
    loops = jnp.arange(n, dtype=edge_index.dtype)
    ei = jnp.concatenate([edge_index, jnp.stack([loops, loops])], axis=1)
    src, dst = ei[0], ei[1]
    pad = jnp.full((_EPAD - src.shape[0],), _PADN, dtype=jnp.int32)
    srcp = jnp.concatenate([src.astype(jnp.int32), pad])
    dstp = jnp.concatenate([dst.astype(jnp.int32), pad])
    z128 = jnp.zeros((_NP, 128), jnp.float32)

    def npad(a):
        return jnp.pad(a, ((0, _NP - n),) + ((0, 0),) * (a.ndim - 1))

    xp = npad(x)
    posp = jnp.pad(pos, ((0, _NP - n), (0, 5)))
    wp8 = jnp.pad(ln1_w[d:], ((0, 5), (0, 0)))
    u, v = _tc_pre(xp, posp, ln1_w[:d], wp8, ln1_b.reshape(1, -1))
    r = _erelu(u, v, srcp, dstp)
    h2 = _tc_edgemm(r, ln2_w)
    agg = jax.ops.segment_max(h2[: src.shape[0]], dst,
                              num_segments=n).astype(jnp.float32) + ln2_b

    xw, a_s, a_d, mx = _tc_mlp(
        npad(agg), gn1_w, gn1_b.reshape(1, -1), gn2_w, gn2_b.reshape(1, -1),
        gn3_w, gn3_b.reshape(1, -1), gat_w,
        gat_asrc.reshape(1, -1), gat_adst.reshape(1, -1))
    t = jnp.max(mx[:, 0]) + jnp.max(mx[:, 1])
    m = jnp.where(t > 0, t, 0.2 * t)

    dh, ae = _gatden(a_s.reshape(-1), a_d.reshape(-1), srcp, dstp,
                     jnp.broadcast_to(m, (16,)), z128)
    denom_full, deg = _tc_den(dh)
    nh = _gatnum(xw, denom_full.reshape(-1), ae, srcp, dstp, z128)
    p, dinv = _tc_gatfin(nh, deg, gat_b.reshape(1, -1))

    qh = _apass(p, srcp, dstp, z128)
    p = _tc_gcn(qh, dinv, gcn1_w, gcn1_b.reshape(1, -1))
    qh = _apass(p, srcp, dstp, z128)
    p = _tc_gcn(qh, dinv, gcn2_w, gcn2_b.reshape(1, -1))
    qh = _apass(p, srcp, dstp, z128)
    return _tc_final(qh, dinv, out_w, out_b.reshape(1, -1))[:n]


# speedup vs baseline: 1.5285x; 1.5285x over previous
import jax
import jax.numpy as jnp
from jax import lax
from jax.experimental import pallas as pl
from jax.experimental.pallas import tpu as pltpu
from jax.experimental.pallas import tpu_sc as plsc

_N = 10000
_NP = 10240
_ECH = 128
_NCH = 82
_EPT = _NCH * _ECH
_EPAD = 32 * _EPT
_PADN = 10008
_MESH = plsc.VectorSubcoreMesh(core_axis_name="c", subcore_axis_name="s")


def _prelude(z_hbm, acc, s, width):
    nps = _NP // 16
    slab = s * nps
    pltpu.sync_copy(z_hbm.at[pl.ds(slab, nps)], acc.at[pl.ds(slab, nps)])
    plsc.subcore_barrier()
    return slab, nps


def _epilogue(acc, out_hbm, c, slab, nps):
    plsc.subcore_barrier()
    pltpu.sync_copy(acc.at[pl.ds(slab, nps)], out_hbm.at[c, pl.ds(slab, nps)])


def _apass_body(p_hbm, src_hbm, dst_hbm, zero_hbm, out_hbm,
                sidx0, sidx1, didx0, didx1, didx2, didx3, rows0, rows1,
                ss0, ss1, sd0, sd1, sd2, sd3, gr0, gr1, ws0, ws1, acc):
    c = lax.axis_index("c")
    s = lax.axis_index("s")
    slab, nps = _prelude(zero_hbm, acc, s, 128)
    base0 = (c * 16 + s) * _EPT
    sidx = (sidx0, sidx1)
    didx = (didx0, didx1, didx2, didx3)
    rows = (rows0, rows1)
    ssem = (ss0, ss1)
    dsem = (sd0, sd1, sd2, sd3)
    rsem = (gr0, gr1)
    wsem = (ws0, ws1)

    def idx_load(k, b2, b4):
        pltpu.async_copy(src_hbm.at[pl.ds(base0 + k * _ECH, _ECH)], sidx[b2], ssem[b2])
        pltpu.async_copy(dst_hbm.at[pl.ds(base0 + k * _ECH, _ECH)], didx[b4], dsem[b4])

    def idx_wait(b2, b4):
        pltpu.make_async_copy(src_hbm.at[pl.ds(0, _ECH)], sidx[b2], ssem[b2]).wait()
        pltpu.make_async_copy(dst_hbm.at[pl.ds(0, _ECH)], didx[b4], dsem[b4]).wait()

    def gath(b2):
        pltpu.async_copy(p_hbm.at[sidx[b2]], rows[b2], rsem[b2])

    def gath_wait(b2):
        pltpu.make_async_copy(p_hbm.at[sidx[b2]], rows[b2], rsem[b2]).wait()

    idx_load(0, 0, 0)
    idx_wait(0, 0)
    gath(0)
    idx_load(1, 1, 1)

    def pair(kk, carry):
        for b in (0, 1):
            k = 2 * kk + b
            nb2 = 1 - b
            gath_wait(b)

            @pl.when(k + 1 < _NCH)
            def _():
                idx_wait(nb2, nb2)
                gath(nb2)

            pltpu.sync_copy(rows[b], acc.at[didx[b]], add=True)

            @pl.when(k + 2 < _NCH)
            def _():
                idx_load(k + 2, b, b)
        return carry

    lax.fori_loop(0, _NCH // 2, pair, 0)
    _epilogue(acc, out_hbm, c, slab, nps)


_apass = pl.kernel(
    _apass_body,
    out_type=jax.ShapeDtypeStruct((2, _NP, 128), jnp.float32),
    mesh=_MESH,
    scratch_types=[
        pltpu.VMEM((_ECH,), jnp.int32), pltpu.VMEM((_ECH,), jnp.int32),
        pltpu.VMEM((_ECH,), jnp.int32), pltpu.VMEM((_ECH,), jnp.int32),
        pltpu.VMEM((_ECH,), jnp.int32), pltpu.VMEM((_ECH,), jnp.int32),
        pltpu.VMEM((_ECH, 128), jnp.float32), pltpu.VMEM((_ECH, 128), jnp.float32),
        pltpu.SemaphoreType.DMA, pltpu.SemaphoreType.DMA,
        pltpu.SemaphoreType.DMA, pltpu.SemaphoreType.DMA,
        pltpu.SemaphoreType.DMA, pltpu.SemaphoreType.DMA,
        pltpu.SemaphoreType.DMA, pltpu.SemaphoreType.DMA,
        pltpu.SemaphoreType.DMA, pltpu.SemaphoreType.DMA,
        pltpu.VMEM_SHARED((_NP, 128), jnp.float32),
    ],
)


def _gatden_body(as_hbm, ad_hbm, src_hbm, dst_hbm, m_hbm, zero_hbm,
                 out_hbm, ae_hbm,
                 sidx0, sidx1, didx0, didx1, asv0, asv1, adv0, adv1,
                 scv, mv,
                 ss0, ss1, sd0, sd1, ga0, ga1, gb0, gb1, acc):
    c = lax.axis_index("c")
    s = lax.axis_index("s")
    slab, nps = _prelude(zero_hbm, acc, s, 128)
    pltpu.sync_copy(m_hbm, mv)
    base0 = (c * 16 + s) * _EPT
    sidx = (sidx0, sidx1)
    didx = (didx0, didx1)
    asv = (asv0, asv1)
    adv = (adv0, adv1)
    ssem = (ss0, ss1)
    dsem = (sd0, sd1)
    asem = (ga0, ga1)
    bsem = (gb0, gb1)
    iota = lax.iota(jnp.int32, 16)
    mvec = mv[...]

    def zrow(e, carry):
        for cc in range(8):
            scv[e, pl.ds(cc * 16, 16)] = jnp.zeros((16,), jnp.float32)
        return carry

    lax.fori_loop(0, _ECH, zrow, 0)

    def idx_load(k, b):
        pltpu.async_copy(src_hbm.at[pl.ds(base0 + k * _ECH, _ECH)], sidx[b], ssem[b])
        pltpu.async_copy(dst_hbm.at[pl.ds(base0 + k * _ECH, _ECH)], didx[b], dsem[b])

    def idx_wait(b):
        pltpu.make_async_copy(src_hbm.at[pl.ds(0, _ECH)], sidx[b], ssem[b]).wait()
        pltpu.make_async_copy(dst_hbm.at[pl.ds(0, _ECH)], didx[b], dsem[b]).wait()

    def gath(b):
        pltpu.async_copy(as_hbm.at[sidx[b]], asv[b], asem[b])
        pltpu.async_copy(ad_hbm.at[didx[b]], adv[b], bsem[b])

    def gath_wait(b):
        pltpu.make_async_copy(as_hbm.at[sidx[b]], asv[b], asem[b]).wait()
        pltpu.make_async_copy(ad_hbm.at[didx[b]], adv[b], bsem[b]).wait()

    idx_load(0, 0)
    idx_wait(0)
    gath(0)
    idx_load(1, 1)

    def pair(kk, carry):
        for b in (0, 1):
            k = 2 * kk + b
            nb = 1 - b
            gath_wait(b)

            @pl.when(k + 1 < _NCH)
            def _():
                idx_wait(nb)
                gath(nb)

            for j in range(_ECH // 16):
                a = asv[b][pl.ds(j * 16, 16)] + adv[b][pl.ds(j * 16, 16)]
                a = jnp.where(a > 0.0, a, 0.2 * a)
                av = jnp.exp(jnp.minimum(a - mvec, 50.0))
                asv[b][pl.ds(j * 16, 16)] = av
                for ee in range(16):
                    scv[j * 16 + ee, pl.ds(0, 16)] = jnp.where(
                        iota == 0, av[ee], jnp.where(iota == 1, 1.0, 0.0))
            pltpu.sync_copy(asv[b], ae_hbm.at[pl.ds(base0 + k * _ECH, _ECH)])
            pltpu.sync_copy(scv, acc.at[didx[b]], add=True)

            @pl.when(k + 2 < _NCH)
            def _():
                idx_load(k + 2, b)
        return carry

    lax.fori_loop(0, _NCH // 2, pair, 0)
    _epilogue(acc, out_hbm, c, slab, nps)


_gatden = pl.kernel(
    _gatden_body,
    out_type=(jax.ShapeDtypeStruct((2, _NP, 128), jnp.float32),
              jax.ShapeDtypeStruct((_EPAD,), jnp.float32)),
    mesh=_MESH,
    scratch_types=[
        pltpu.VMEM((_ECH,), jnp.int32), pltpu.VMEM((_ECH,), jnp.int32),
        pltpu.VMEM((_ECH,), jnp.int32), pltpu.VMEM((_ECH,), jnp.int32),
        pltpu.VMEM((_ECH,), jnp.float32), pltpu.VMEM((_ECH,), jnp.float32),
        pltpu.VMEM((_ECH,), jnp.float32), pltpu.VMEM((_ECH,), jnp.float32),
        pltpu.VMEM((_ECH, 128), jnp.float32),
        pltpu.VMEM((16,), jnp.float32),
        pltpu.SemaphoreType.DMA, pltpu.SemaphoreType.DMA,
        pltpu.SemaphoreType.DMA, pltpu.SemaphoreType.DMA,
        pltpu.SemaphoreType.DMA, pltpu.SemaphoreType.DMA,
        pltpu.SemaphoreType.DMA, pltpu.SemaphoreType.DMA,
        pltpu.VMEM_SHARED((_NP, 128), jnp.float32),
    ],
)


def _gatnum_body(xw_hbm, den_hbm, ae_hbm, src_hbm, dst_hbm, zero_hbm, out_hbm,
                 sidx0, sidx1, didx0, didx1, aev0, aev1, dnv0, dnv1,
                 rows0, rows1,
                 ss0, ss1, sd0, sd1, ga0, ga1, gb0, gb1, gr0, gr1, acc):
    c = lax.axis_index("c")
    s = lax.axis_index("s")
    slab, nps = _prelude(zero_hbm, acc, s, 128)
    base0 = (c * 16 + s) * _EPT
    sidx = (sidx0, sidx1)
    didx = (didx0, didx1)
    aev = (aev0, aev1)
    dnv = (dnv0, dnv1)
    rows = (rows0, rows1)
    ssem = (ss0, ss1)
    dsem = (sd0, sd1)
    asem = (ga0, ga1)
    bsem = (gb0, gb1)
    rsem = (gr0, gr1)

    def idx_load(k, b):
        pltpu.async_copy(src_hbm.at[pl.ds(base0 + k * _ECH, _ECH)], sidx[b], ssem[b])
        pltpu.async_copy(dst_hbm.at[pl.ds(base0 + k * _ECH, _ECH)], didx[b], dsem[b])

    def idx_wait(b):
        pltpu.make_async_copy(src_hbm.at[pl.ds(0, _ECH)], sidx[b], ssem[b]).wait()
        pltpu.make_async_copy(dst_hbm.at[pl.ds(0, _ECH)], didx[b], dsem[b]).wait()

    def gath(k, b):
        pltpu.async_copy(ae_hbm.at[pl.ds(base0 + k * _ECH, _ECH)], aev[b], asem[b])
        pltpu.async_copy(den_hbm.at[didx[b]], dnv[b], bsem[b])
        pltpu.async_copy(xw_hbm.at[sidx[b]], rows[b], rsem[b])

    def gath_wait(b):
        pltpu.make_async_copy(ae_hbm.at[pl.ds(0, _ECH)], aev[b], asem[b]).wait()
        pltpu.make_async_copy(den_hbm.at[didx[b]], dnv[b], bsem[b]).wait()
        pltpu.make_async_copy(xw_hbm.at[sidx[b]], rows[b], rsem[b]).wait()

    idx_load(0, 0)
    idx_wait(0)
    gath(0, 0)
    idx_load(1, 1)

    def pair(kk, carry):
        for b in (0, 1):
            k = 2 * kk + b
            nb = 1 - b
            gath_wait(b)

            @pl.when(k + 1 < _NCH)
            def _():
                idx_wait(nb)
                gath(k + 1, nb)

            def grp(j, carry2):
                av = aev[b][pl.ds(j * 16, 16)] / dnv[b][pl.ds(j * 16, 16)]
                for ee in range(16):
                    e = j * 16 + ee
                    w = av[ee]
                    for cc in range(8):
                        rows[b][e, pl.ds(cc * 16, 16)] = (
                            rows[b][e, pl.ds(cc * 16, 16)] * w)
                return carry2

            lax.fori_loop(0, _ECH // 16, grp, 0)
            pltpu.sync_copy(rows[b], acc.at[didx[b]], add=True)

            @pl.when(k + 2 < _NCH)
            def _():
                idx_load(k + 2, b)
        return carry

    lax.fori_loop(0, _NCH // 2, pair, 0)
    _epilogue(acc, out_hbm, c, slab, nps)


_gatnum = pl.kernel(
    _gatnum_body,
    out_type=jax.ShapeDtypeStruct((2, _NP, 128), jnp.float32),
    mesh=_MESH,
    scratch_types=[
        pltpu.VMEM((_ECH,), jnp.int32), pltpu.VMEM((_ECH,), jnp.int32),
        pltpu.VMEM((_ECH,), jnp.int32), pltpu.VMEM((_ECH,), jnp.int32),
        pltpu.VMEM((_ECH,), jnp.float32), pltpu.VMEM((_ECH,), jnp.float32),
        pltpu.VMEM((_ECH,), jnp.float32), pltpu.VMEM((_ECH,), jnp.float32),
        pltpu.VMEM((_ECH, 128), jnp.float32), pltpu.VMEM((_ECH, 128), jnp.float32),
        pltpu.SemaphoreType.DMA, pltpu.SemaphoreType.DMA,
        pltpu.SemaphoreType.DMA, pltpu.SemaphoreType.DMA,
        pltpu.SemaphoreType.DMA, pltpu.SemaphoreType.DMA,
        pltpu.SemaphoreType.DMA, pltpu.SemaphoreType.DMA,
        pltpu.SemaphoreType.DMA, pltpu.SemaphoreType.DMA,
        pltpu.VMEM_SHARED((_NP, 128), jnp.float32),
    ],
)


def _erelu_body(u_hbm, v_hbm, src_hbm, dst_hbm, r_hbm,
                sidx0, sidx1, didx0, didx1, ru0, ru1, rv0, rv1,
                ss0, ss1, sd0, sd1, gu0, gu1, gv0, gv1):
    c = lax.axis_index("c")
    s = lax.axis_index("s")
    base0 = (c * 16 + s) * _EPT
    sidx = (sidx0, sidx1)
    didx = (didx0, didx1)
    ru = (ru0, ru1)
    rv = (rv0, rv1)
    ssem = (ss0, ss1)
    dsem = (sd0, sd1)
    usem = (gu0, gu1)
    vsem = (gv0, gv1)

    def idx_load(k, b):
        pltpu.async_copy(src_hbm.at[pl.ds(base0 + k * _ECH, _ECH)], sidx[b], ssem[b])
        pltpu.async_copy(dst_hbm.at[pl.ds(base0 + k * _ECH, _ECH)], didx[b], dsem[b])

    def idx_wait(b):
        pltpu.make_async_copy(src_hbm.at[pl.ds(0, _ECH)], sidx[b], ssem[b]).wait()
        pltpu.make_async_copy(dst_hbm.at[pl.ds(0, _ECH)], didx[b], dsem[b]).wait()

    def gath(b):
        pltpu.async_copy(u_hbm.at[sidx[b]], ru[b], usem[b])
        pltpu.async_copy(v_hbm.at[didx[b]], rv[b], vsem[b])

    def gath_wait(b):
        pltpu.make_async_copy(u_hbm.at[sidx[b]], ru[b], usem[b]).wait()
        pltpu.make_async_copy(v_hbm.at[didx[b]], rv[b], vsem[b]).wait()

    idx_load(0, 0)
    idx_wait(0)
    gath(0)
    idx_load(1, 1)

    def pair(kk, carry):
        for b in (0, 1):
            k = 2 * kk + b
            nb = 1 - b
            gath_wait(b)

            @pl.when(k + 1 < _NCH)
            def _():
                idx_wait(nb)
                gath(nb)

            def erow(e, carry2):
                for cc in range(8):
                    d = ru[b][e, pl.ds(cc * 16, 16)] - rv[b][e, pl.ds(cc * 16, 16)]
                    ru[b][e, pl.ds(cc * 16, 16)] = jnp.maximum(d, 0.0)
                return carry2

            lax.fori_loop(0, _ECH, erow, 0)
            pltpu.sync_copy(ru[b], r_hbm.at[pl.ds(base0 + k * _ECH, _ECH)])

            @pl.when(k + 2 < _NCH)
            def _():
                idx_load(k + 2, b)
        return carry

    lax.fori_loop(0, _NCH // 2, pair, 0)


_erelu = pl.kernel(
    _erelu_body,
    out_type=jax.ShapeDtypeStruct((_EPAD, 128), jnp.float32),
    mesh=_MESH,
    scratch_types=[
        pltpu.VMEM((_ECH,), jnp.int32), pltpu.VMEM((_ECH,), jnp.int32),
        pltpu.VMEM((_ECH,), jnp.int32), pltpu.VMEM((_ECH,), jnp.int32),
        pltpu.VMEM((_ECH, 128), jnp.float32), pltpu.VMEM((_ECH, 128), jnp.float32),
        pltpu.VMEM((_ECH, 128), jnp.float32), pltpu.VMEM((_ECH, 128), jnp.float32),
        pltpu.SemaphoreType.DMA, pltpu.SemaphoreType.DMA,
        pltpu.SemaphoreType.DMA, pltpu.SemaphoreType.DMA,
        pltpu.SemaphoreType.DMA, pltpu.SemaphoreType.DMA,
        pltpu.SemaphoreType.DMA, pltpu.SemaphoreType.DMA,
    ],
)


_RB = 1024
_EB = 2048


def _rows_spec(width):
    return pl.BlockSpec((_RB, width), lambda i: (i, 0))


def _full_spec(r, c):
    return pl.BlockSpec((r, c), lambda i: (0, 0))


def _halves_spec(width):
    return pl.BlockSpec((2, _RB, width), lambda i: (0, i, 0))


def _tc_pre_body(x_ref, p_ref, wx_ref, wp_ref, b_ref, u_ref, v_ref):
    vv = jnp.dot(p_ref[...], wp_ref[...], preferred_element_type=jnp.float32)
    u_ref[...] = (jnp.dot(x_ref[...], wx_ref[...],
                          preferred_element_type=jnp.float32)
                  + vv + b_ref[...])
    v_ref[...] = vv


def _tc_pre(xp, posp, wx, wp, b1):
    return pl.pallas_call(
        _tc_pre_body,
        grid=(_NP // _RB,),
        in_specs=[_rows_spec(128), _rows_spec(8), _full_spec(128, 128),
                  _full_spec(8, 128), _full_spec(1, 128)],
        out_specs=[_rows_spec(128), _rows_spec(128)],
        out_shape=[jax.ShapeDtypeStruct((_NP, 128), jnp.float32),
                   jax.ShapeDtypeStruct((_NP, 128), jnp.float32)],
    )(xp, posp, wx, wp, b1)


def _tc_edgemm_body(r_ref, w_ref, o_ref):
    o_ref[...] = jnp.dot(r_ref[...], w_ref[...],
                         preferred_element_type=jnp.float32).astype(jnp.bfloat16)


def _tc_edgemm(r, w):
    return pl.pallas_call(
        _tc_edgemm_body,
        grid=(_EPAD // _EB,),
        in_specs=[pl.BlockSpec((_EB, 128), lambda i: (i, 0)),
                  _full_spec(128, 128)],
        out_specs=pl.BlockSpec((_EB, 128), lambda i: (i, 0)),
        out_shape=jax.ShapeDtypeStruct((_EPAD, 128), jnp.bfloat16),
    )(r, w)


def _tc_mlp_body(agg_ref, w1_ref, b1_ref, w2_ref, b2_ref, w3_ref, b3_ref,
                 wg_ref, asr_ref, adr_ref, xw_ref, as_ref, ad_ref, mx_ref):
    g = jax.nn.relu(jnp.dot(agg_ref[...], w1_ref[...],
                            preferred_element_type=jnp.float32) + b1_ref[...])
    g = jax.nn.relu(jnp.dot(g, w2_ref[...],
                            preferred_element_type=jnp.float32) + b2_ref[...])
    h0 = jnp.dot(g, w3_ref[...], preferred_element_type=jnp.float32) + b3_ref[...]
    xw = jnp.dot(h0, wg_ref[...], preferred_element_type=jnp.float32)
    xw_ref[...] = xw
    a_s = jnp.sum(xw * asr_ref[...], axis=1, keepdims=True)
    a_d = jnp.sum(xw * adr_ref[...], axis=1, keepdims=True)
    as_ref[...] = a_s
    ad_ref[...] = a_d
    col = lax.broadcasted_iota(jnp.int32, (8, 128), 1)
    mx_ref[...] = jnp.where(col == 0, jnp.max(a_s),
                            jnp.where(col == 1, jnp.max(a_d), 0.0))


def _tc_mlp(aggp, w1, b1, w2, b2, w3, b3, wg, asr, adr):
    return pl.pallas_call(
        _tc_mlp_body,
        grid=(_NP // _RB,),
        in_specs=[_rows_spec(128), _full_spec(128, 64), _full_spec(1, 64),
                  _full_spec(64, 256), _full_spec(1, 256),
                  _full_spec(256, 128), _full_spec(1, 128),
                  _full_spec(128, 128), _full_spec(1, 128), _full_spec(1, 128)],
        out_specs=[_rows_spec(128),
                   pl.BlockSpec((_RB, 1), lambda i: (i, 0)),
                   pl.BlockSpec((_RB, 1), lambda i: (i, 0)),
                   pl.BlockSpec((8, 128), lambda i: (i, 0))],
        out_shape=[jax.ShapeDtypeStruct((_NP, 128), jnp.float32),
                   jax.ShapeDtypeStruct((_NP, 1), jnp.float32),
                   jax.ShapeDtypeStruct((_NP, 1), jnp.float32),
                   jax.ShapeDtypeStruct((_NP // _RB * 8, 128), jnp.float32)],
    )(aggp, w1, b1, w2, b2, w3, b3, wg, asr, adr)


def _tc_den_body(dh_ref, den_ref, deg_ref):
    s = dh_ref[0] + dh_ref[1]
    den_ref[...] = s[:, 0:1]
    deg_ref[...] = s[:, 1:2]


def _tc_den(dh):
    return pl.pallas_call(
        _tc_den_body,
        grid=(_NP // _RB,),
        in_specs=[_halves_spec(128)],
        out_specs=[pl.BlockSpec((_RB, 1), lambda i: (i, 0)),
                   pl.BlockSpec((_RB, 1), lambda i: (i, 0))],
        out_shape=[jax.ShapeDtypeStruct((_NP, 1), jnp.float32),
                   jax.ShapeDtypeStruct((_NP, 1), jnp.float32)],
    )(dh)


def _tc_gatfin_body(nh_ref, deg_ref, b_ref, p_ref, dinv_ref):
    h1 = jax.nn.relu(nh_ref[0] + nh_ref[1] + b_ref[...])
    deg = deg_ref[...]
    dinv = jnp.where(deg > 0.0, lax.rsqrt(jnp.maximum(deg, 1e-30)), 0.0)
    dinv_ref[...] = dinv
    p_ref[...] = dinv * h1


def _tc_gatfin(nh, deg, gb):
    return pl.pallas_call(
        _tc_gatfin_body,
        grid=(_NP // _RB,),
        in_specs=[_halves_spec(128), pl.BlockSpec((_RB, 1), lambda i: (i, 0)),
                  _full_spec(1, 128)],
        out_specs=[_rows_spec(128), pl.BlockSpec((_RB, 1), lambda i: (i, 0))],
        out_shape=[jax.ShapeDtypeStruct((_NP, 128), jnp.float32),
                   jax.ShapeDtypeStruct((_NP, 1), jnp.float32)],
    )(nh, deg, gb)


def _tc_gcn_body(qh_ref, dinv_ref, w_ref, b_ref, p_ref):
    q = (qh_ref[0] + qh_ref[1]) * dinv_ref[...]
    hnew = jax.nn.relu(jnp.dot(q, w_ref[...],
                               preferred_element_type=jnp.float32) + b_ref[...])
    p_ref[...] = dinv_ref[...] * hnew


def _tc_gcn(qh, dinv, w, b):
    return pl.pallas_call(
        _tc_gcn_body,
        grid=(_NP // _RB,),
        in_specs=[_halves_spec(128), pl.BlockSpec((_RB, 1), lambda i: (i, 0)),
                  _full_spec(128, 128), _full_spec(1, 128)],
        out_specs=_rows_spec(128),
        out_shape=jax.ShapeDtypeStruct((_NP, 128), jnp.float32),
    )(qh, dinv, w, b)


def _tc_final_body(qh_ref, dinv_ref, w_ref, b_ref, o_ref):
    q = (qh_ref[0] + qh_ref[1]) * dinv_ref[...]
    o_ref[...] = jnp.dot(q, w_ref[...],
                         preferred_element_type=jnp.float32) + b_ref[...]


def _tc_final(qh, dinv, w, b):
    return pl.pallas_call(
        _tc_final_body,
        grid=(_NP // _RB,),
        in_specs=[_halves_spec(128), pl.BlockSpec((_RB, 1), lambda i: (i, 0)),
                  _full_spec(128, 128), _full_spec(1, 128)],
        out_specs=_rows_spec(128),
        out_shape=jax.ShapeDtypeStruct((_NP, 128), jnp.float32),
    )(qh, dinv, w, b)


def kernel(x, pos, edge_index, ln1_w, ln1_b, ln2_w, ln2_b, gn1_w, gn1_b, gn2_w, gn2_b, gn3_w, gn3_b, gat_w, gat_asrc, gat_adst, gat_b, gcn1_w, gcn1_b, gcn2_w, gcn2_b, out_w, out_b):
    n, d = x.shape
    loops = jnp.arange(n, dtype=edge_index.dtype)
    ei = jnp.concatenate([edge_index, jnp.stack([loops, loops])], axis=1)
    src, dst = ei[0], ei[1]
    pad = jnp.full((_EPAD - src.shape[0],), _PADN, dtype=jnp.int32)
    srcp = jnp.concatenate([src.astype(jnp.int32), pad])
    dstp = jnp.concatenate([dst.astype(jnp.int32), pad])
    z128 = jnp.zeros((_NP, 128), jnp.float32)

    def npad(a):
        return jnp.pad(a, ((0, _NP - n),) + ((0, 0),) * (a.ndim - 1))

    xp = npad(x)
    posp = jnp.pad(pos, ((0, _NP - n), (0, 5)))
    wp8 = jnp.pad(ln1_w[d:], ((0, 5), (0, 0)))
    u, v = _tc_pre(xp, posp, ln1_w[:d], wp8, ln1_b.reshape(1, -1))
    r = _erelu(u, v, srcp, dstp)
    h2 = _tc_edgemm(r, ln2_w)
    agg = jax.ops.segment_max(h2[: src.shape[0]], dst,
                              num_segments=n).astype(jnp.float32) + ln2_b

    xw, a_s, a_d, mx = _tc_mlp(
        npad(agg), gn1_w, gn1_b.reshape(1, -1), gn2_w, gn2_b.reshape(1, -1),
        gn3_w, gn3_b.reshape(1, -1), gat_w,
        gat_asrc.reshape(1, -1), gat_adst.reshape(1, -1))
    t = jnp.max(mx[:, 0]) + jnp.max(mx[:, 1])
    m = jnp.where(t > 0, t, 0.2 * t)

    dh, ae = _gatden(a_s.reshape(-1), a_d.reshape(-1), srcp, dstp,
                     jnp.broadcast_to(m, (16,)), z128)
    denom_full, deg = _tc_den(dh)
    nh = _gatnum(xw, denom_full.reshape(-1), ae, srcp, dstp, z128)
    p, dinv = _tc_gatfin(nh, deg, gat_b.reshape(1, -1))

    qh = _apass(p, srcp, dstp, z128)
    p = _tc_gcn(qh, dinv, gcn1_w, gcn1_b.reshape(1, -1))
    qh = _apass(p, srcp, dstp, z128)
    p = _tc_gcn(qh, dinv, gcn2_w, gcn2_b.reshape(1, -1))
    qh = _apass(p, srcp, dstp, z128)
    return _tc_final(qh, dinv, out_w, out_b.reshape(1, -1))[:n]
